# baseline (speedup 1.0000x reference)
import functools

import jax
import jax.numpy as jnp
from jax import lax
from jax.experimental import pallas as pl
from jax.experimental.pallas import tpu as pltpu
from jax.experimental.pallas import tpu_sc as plsc

VOCAB = 100000
EMB = 32
BT = 1024
VT = 4096
NV = (VOCAB + VT - 1) // VT
EDGE = VOCAB - (NV - 1) * VT
NBUF = 2
NSPLIT = 4
RS = BT // NSPLIT


def _make_sc_gather(V, D, B):
    info = plsc.get_sparse_core_info()
    nc, ns = info.num_cores, info.num_subcores
    nw = nc * ns
    b_per_w = B // nw
    mesh = plsc.VectorSubcoreMesh(core_axis_name="c", subcore_axis_name="s")

    @functools.partial(
        pl.kernel,
        mesh=mesh,
        compiler_params=pltpu.CompilerParams(use_tc_tiling_on_sc=False),
        out_type=jax.ShapeDtypeStruct((B, D), jnp.float32),
        scratch_types=[
            pltpu.VMEM((b_per_w,), jnp.int32),
            pltpu.VMEM((b_per_w, D), jnp.float32),
            pltpu.SemaphoreType.DMA,
        ],
    )
    def gather(table_hbm, idx_hbm, out_hbm, idx_v, rows_v, sem):
        wid = lax.axis_index("s") * nc + lax.axis_index("c")
        base = wid * b_per_w
        pltpu.sync_copy(idx_hbm.at[pl.ds(base, b_per_w)], idx_v)
        pltpu.async_copy(table_hbm.at[idx_v], rows_v, sem).wait()
        pltpu.sync_copy(rows_v, out_hbm.at[pl.ds(base, b_per_w)])

    return gather


CH = 1024
NCH = VT // CH
EREM = EDGE - CH


def _chunk_copy(j, c, buf_ref, out_ref, sem_ref):
    slot = lax.rem(j, NBUF)
    return pltpu.make_async_copy(
        buf_ref.at[slot, :, pl.ds(c * CH, CH)],
        out_ref.at[:, pl.ds(j * VT + c * CH, CH)],
        sem_ref.at[slot, c])


def _edge_copy(ebuf_ref, out_ref, esem_ref):
    return pltpu.make_async_copy(
        ebuf_ref,
        out_ref.at[:, pl.ds(VOCAB - EREM, EREM)],
        esem_ref)


def _stats_update(xc, c0, j, t_ref, m_ref, s_ref, g_ref, valid):
    li = lax.broadcasted_iota(jnp.int32, (BT, CH), 1)
    if valid < CH:
        xm = jnp.where(li < valid, xc, -jnp.inf)
    else:
        xm = xc
    m_old = m_ref[...]
    m_new = jnp.maximum(m_old, jnp.max(xm, axis=1, keepdims=True))
    s_ref[...] = (s_ref[...] * jnp.exp(m_old - m_new)
                  + jnp.sum(jnp.exp(xm - m_new), axis=1, keepdims=True))
    m_ref[...] = m_new
    g_ref[...] += jnp.sum(jnp.where(li == t_ref[...] - (j * VT + c0),
                                    xc, 0.0), axis=1, keepdims=True)


def _logits_loss_body(emb_ref, w_ref, b_ref, t_ref, out_ref, loss_ref,
                      buf_ref, ebuf_ref, m_ref, s_ref, g_ref,
                      sem_ref, esem_ref):
    j = pl.program_id(0)
    slot = lax.rem(j, NBUF)

    @pl.when(j == 0)
    def _init():
        m_ref[...] = jnp.full_like(m_ref, -jnp.inf)
        s_ref[...] = jnp.zeros_like(s_ref)
        g_ref[...] = jnp.zeros_like(g_ref)

    @pl.when(j >= NBUF)
    def _wait_prev():
        for c in range(NCH):
            _chunk_copy(j - NBUF, c, buf_ref, out_ref, sem_ref).wait()

    @pl.when(j < NV - 1)
    def _full_tile():
        for c in range(NCH):
            sl = pl.ds(c * CH, CH)
            xc = jnp.dot(emb_ref[...], w_ref[:, sl],
                         preferred_element_type=jnp.float32) + b_ref[:, sl]
            buf_ref[slot, :, sl] = xc
            _chunk_copy(j, c, buf_ref, out_ref, sem_ref).start()
            _stats_update(xc, c * CH, j, t_ref, m_ref, s_ref, g_ref, CH)

    @pl.when(j == NV - 1)
    def _edge_tile():
        sl = pl.ds(0, CH)
        xc = jnp.dot(emb_ref[...], w_ref[:, sl],
                     preferred_element_type=jnp.float32) + b_ref[:, sl]
        buf_ref[slot, :, sl] = xc
        _chunk_copy(j, 0, buf_ref, out_ref, sem_ref).start()
        _stats_update(xc, 0, j, t_ref, m_ref, s_ref, g_ref, CH)
        sl = pl.ds(CH, CH)
        xc = jnp.dot(emb_ref[...], w_ref[:, sl],
                     preferred_element_type=jnp.float32) + b_ref[:, sl]
        ebuf_ref[...] = xc[:, :EREM]
        _edge_copy(ebuf_ref, out_ref, esem_ref).start()
        _stats_update(xc, CH, j, t_ref, m_ref, s_ref, g_ref, EREM)
        for c in range(NCH):
            _chunk_copy(j - 1, c, buf_ref, out_ref, sem_ref).wait()
        _chunk_copy(j, 0, buf_ref, out_ref, sem_ref).wait()
        _edge_copy(ebuf_ref, out_ref, esem_ref).wait()
        nll = m_ref[...] + jnp.log(s_ref[...]) - g_ref[...]
        loss_ref[0, 0] = jnp.sum(nll) * (1.0 / BT)


def _logits_and_loss(emb, W, b2, tflat):
    return pl.pallas_call(
        _logits_loss_body,
        grid=(NV,),
        in_specs=[
            pl.BlockSpec((BT, EMB), lambda j: (0, 0)),
            pl.BlockSpec((EMB, VT), lambda j: (0, j)),
            pl.BlockSpec((1, VT), lambda j: (0, j)),
            pl.BlockSpec((BT, 1), lambda j: (0, 0)),
        ],
        out_specs=[
            pl.BlockSpec(memory_space=pl.ANY),
            pl.BlockSpec(memory_space=pltpu.SMEM),
        ],
        out_shape=[
            jax.ShapeDtypeStruct((BT, VOCAB), jnp.float32),
            jax.ShapeDtypeStruct((1, 1), jnp.float32),
        ],
        scratch_shapes=[
            pltpu.VMEM((NBUF, BT, VT), jnp.float32),
            pltpu.VMEM((BT, EREM), jnp.float32),
            pltpu.VMEM((BT, 1), jnp.float32),
            pltpu.VMEM((BT, 1), jnp.float32),
            pltpu.VMEM((BT, 1), jnp.float32),
            pltpu.SemaphoreType.DMA((NBUF, NCH)),
            pltpu.SemaphoreType.DMA,
        ],
    )(emb, W, b2, tflat)


_sc_gather_cache = []


def _sc_gather(table, idx_flat):
    if not _sc_gather_cache:
        _sc_gather_cache.append(_make_sc_gather(VOCAB, EMB, BT))
    return _sc_gather_cache[0](table, idx_flat)


def kernel(idx, targets, token_table, W, b):
    idx_flat = idx.reshape(BT).astype(jnp.int32)
    tflat = targets.reshape(BT, 1).astype(jnp.int32)
    emb = _sc_gather(token_table, idx_flat)
    logits, loss = _logits_and_loss(emb, W, b.reshape(1, VOCAB), tflat)
    return logits, loss[0, 0]


# speedup vs baseline: 1.0599x; 1.0599x over previous
import functools

import jax
import jax.numpy as jnp
from jax import lax
from jax.experimental import pallas as pl
from jax.experimental.pallas import tpu as pltpu
from jax.experimental.pallas import tpu_sc as plsc

VOCAB = 100000
EMB = 32
BT = 1024
VT = 4096
NV = (VOCAB + VT - 1) // VT
NR = 4
RS = BT // NR


def _make_sc_gather(V, D, B):
    info = plsc.get_sparse_core_info()
    nc, ns = info.num_cores, info.num_subcores
    nw = nc * ns
    b_per_w = B // nw
    mesh = plsc.VectorSubcoreMesh(core_axis_name="c", subcore_axis_name="s")

    @functools.partial(
        pl.kernel,
        mesh=mesh,
        compiler_params=pltpu.CompilerParams(use_tc_tiling_on_sc=False),
        out_type=jax.ShapeDtypeStruct((B, D), jnp.float32),
        scratch_types=[
            pltpu.VMEM((b_per_w,), jnp.int32),
            pltpu.VMEM((b_per_w, D), jnp.float32),
            pltpu.SemaphoreType.DMA,
        ],
    )
    def gather(table_hbm, idx_hbm, out_hbm, idx_v, rows_v, sem):
        wid = lax.axis_index("s") * nc + lax.axis_index("c")
        base = wid * b_per_w
        pltpu.sync_copy(idx_hbm.at[pl.ds(base, b_per_w)], idx_v)
        pltpu.async_copy(table_hbm.at[idx_v], rows_v, sem).wait()
        pltpu.sync_copy(rows_v, out_hbm.at[pl.ds(base, b_per_w)])

    return gather


def _logits_loss_body(emb_ref, w_ref, b_ref, t_ref, out_ref, loss_ref,
                      m_ref, s_ref, g_ref):
    j = pl.program_id(0)
    r = pl.program_id(1)
    srow = pl.ds(r * RS, RS)

    @pl.when(j == 0)
    def _init():
        m_ref[srow] = jnp.full((RS, 1), -jnp.inf, jnp.float32)
        s_ref[srow] = jnp.zeros((RS, 1), jnp.float32)
        g_ref[srow] = jnp.zeros((RS, 1), jnp.float32)

    x = jnp.dot(emb_ref[...], w_ref[...],
                preferred_element_type=jnp.float32) + b_ref[...]
    out_ref[...] = x

    li = lax.broadcasted_iota(jnp.int32, (RS, VT), 1)
    bound = jnp.minimum(VOCAB - j * VT, VT)
    xm = jnp.where(li < bound, x, -jnp.inf)
    m_old = m_ref[srow]
    m_new = jnp.maximum(m_old, jnp.max(xm, axis=1, keepdims=True))
    s_ref[srow] = (s_ref[srow] * jnp.exp(m_old - m_new)
                   + jnp.sum(jnp.exp(xm - m_new), axis=1, keepdims=True))
    m_ref[srow] = m_new
    g_ref[srow] += jnp.sum(jnp.where(li == t_ref[...] - j * VT, x, 0.0),
                           axis=1, keepdims=True)

    @pl.when((j == NV - 1) & (r == NR - 1))
    def _fin():
        nll = m_ref[...] + jnp.log(s_ref[...]) - g_ref[...]
        loss_ref[0, 0] = jnp.sum(nll) * (1.0 / BT)


def _logits_and_loss(emb, W, b2, tflat):
    return pl.pallas_call(
        _logits_loss_body,
        grid=(NV, NR),
        in_specs=[
            pl.BlockSpec((RS, EMB), lambda j, r: (r, 0)),
            pl.BlockSpec((EMB, VT), lambda j, r: (0, j)),
            pl.BlockSpec((1, VT), lambda j, r: (0, j)),
            pl.BlockSpec((RS, 1), lambda j, r: (r, 0)),
        ],
        out_specs=[
            pl.BlockSpec((RS, VT), lambda j, r: (r, j)),
            pl.BlockSpec(memory_space=pltpu.SMEM),
        ],
        out_shape=[
            jax.ShapeDtypeStruct((BT, VOCAB), jnp.float32),
            jax.ShapeDtypeStruct((1, 1), jnp.float32),
        ],
        scratch_shapes=[
            pltpu.VMEM((BT, 1), jnp.float32),
            pltpu.VMEM((BT, 1), jnp.float32),
            pltpu.VMEM((BT, 1), jnp.float32),
        ],
    )(emb, W, b2, tflat)


_sc_gather_cache = []


def _sc_gather(table, idx_flat):
    if not _sc_gather_cache:
        _sc_gather_cache.append(_make_sc_gather(VOCAB, EMB, BT))
    return _sc_gather_cache[0](table, idx_flat)


def kernel(idx, targets, token_table, W, b):
    idx_flat = idx.reshape(BT).astype(jnp.int32)
    tflat = targets.reshape(BT, 1).astype(jnp.int32)
    emb = _sc_gather(token_table, idx_flat)
    logits, loss = _logits_and_loss(emb, W, b.reshape(1, VOCAB), tflat)
    return logits, loss[0, 0]


# speedup vs baseline: 1.1078x; 1.0452x over previous
import functools

import jax
import jax.numpy as jnp
from jax import lax
from jax.experimental import pallas as pl
from jax.experimental.pallas import tpu as pltpu
from jax.experimental.pallas import tpu_sc as plsc

VOCAB = 100000
EMB = 32
B, T = 128, 8
BT = B * T
VT = 4096
NV = (VOCAB + VT - 1) // VT


def _make_sc_gather():
    info = plsc.get_sparse_core_info()
    nc, ns = info.num_cores, info.num_subcores
    nw = nc * ns
    rows_per_w = B // nw
    b_per_w = BT // nw
    mesh = plsc.VectorSubcoreMesh(core_axis_name="c", subcore_axis_name="s")

    @functools.partial(
        pl.kernel,
        mesh=mesh,
        compiler_params=pltpu.CompilerParams(use_tc_tiling_on_sc=False),
        out_type=jax.ShapeDtypeStruct((BT, EMB), jnp.float32),
        scratch_types=[
            pltpu.VMEM((b_per_w,), jnp.int32),
            pltpu.VMEM((b_per_w, EMB), jnp.float32),
            pltpu.SemaphoreType.DMA,
        ],
    )
    def gather(table_hbm, idx_hbm, out_hbm, idx_v, rows_v, sem):
        wid = lax.axis_index("s") * nc + lax.axis_index("c")
        for rr in range(rows_per_w):
            pltpu.sync_copy(idx_hbm.at[wid * rows_per_w + rr],
                            idx_v.at[pl.ds(rr * T, T)])
        pltpu.async_copy(table_hbm.at[idx_v], rows_v, sem).wait()
        pltpu.sync_copy(rows_v, out_hbm.at[pl.ds(wid * b_per_w, b_per_w)])

    return gather


def _logits_loss_body(emb_ref, w_ref, b_ref, t_ref, out_ref, loss_ref,
                      m_ref, s_ref, g_ref):
    j = pl.program_id(0)

    @pl.when(j == 0)
    def _init():
        m_ref[...] = jnp.full((BT, 1), -jnp.inf, jnp.float32)
        s_ref[...] = jnp.zeros((BT, 1), jnp.float32)
        g_ref[...] = jnp.zeros((BT, 1), jnp.float32)

    x = jnp.dot(emb_ref[...], w_ref[...],
                preferred_element_type=jnp.float32) + b_ref[...]
    out_ref[...] = x

    tcol = t_ref[...]
    li = lax.broadcasted_iota(jnp.int32, (BT, VT), 1)
    bound = jnp.minimum(VOCAB - j * VT, VT)
    xm = jnp.where(li < bound, x, -jnp.inf)
    m_old = m_ref[...]
    m_new = jnp.maximum(m_old, jnp.max(xm, axis=1, keepdims=True))
    s_ref[...] = (s_ref[...] * jnp.exp(m_old - m_new)
                  + jnp.sum(jnp.exp(xm - m_new), axis=1, keepdims=True))
    m_ref[...] = m_new
    g_ref[...] += jnp.sum(jnp.where(li == tcol - j * VT, x, 0.0),
                          axis=1, keepdims=True)

    @pl.when(j == NV - 1)
    def _fin():
        nll = m_ref[...] + jnp.log(s_ref[...]) - g_ref[...]
        loss_ref[0, 0] = jnp.sum(nll) * (1.0 / BT)


def _logits_and_loss(emb, W, b, targets):
    return pl.pallas_call(
        _logits_loss_body,
        grid=(NV,),
        in_specs=[
            pl.BlockSpec((BT, EMB), lambda j: (0, 0)),
            pl.BlockSpec((EMB, VT), lambda j: (0, j)),
            pl.BlockSpec((VT,), lambda j: (j,)),
            pl.BlockSpec((BT, 1), lambda j: (0, 0)),
        ],
        out_specs=[
            pl.BlockSpec((BT, VT), lambda j: (0, j)),
            pl.BlockSpec(memory_space=pltpu.SMEM),
        ],
        out_shape=[
            jax.ShapeDtypeStruct((BT, VOCAB), jnp.float32),
            jax.ShapeDtypeStruct((1, 1), jnp.float32),
        ],
        scratch_shapes=[
            pltpu.VMEM((BT, 1), jnp.float32),
            pltpu.VMEM((BT, 1), jnp.float32),
            pltpu.VMEM((BT, 1), jnp.float32),
        ],
    )(emb, W, b, targets)


_sc_gather_cache = []


def _sc_gather(table, idx):
    if not _sc_gather_cache:
        _sc_gather_cache.append(_make_sc_gather())
    return _sc_gather_cache[0](table, idx)


def kernel(idx, targets, token_table, W, b):
    emb = _sc_gather(token_table, idx.astype(jnp.int32))
    tflat = targets.reshape(BT, 1).astype(jnp.int32)
    logits, loss = _logits_and_loss(emb, W, b, tflat)
    return logits, loss[0, 0]
